# baseline (speedup 1.0000x reference)
import functools

import jax
import jax.numpy as jnp
from jax import lax
from jax.experimental import pallas as pl
from jax.experimental.pallas import tpu as pltpu
from jax.experimental.pallas import tpu_sc as plsc

N_NODES = 10000
N_PAD = 10240
N_EDGES = 320000
E_PAD = 327680
E_ROWS = 2560
W_PER_SUB = 80
ROWS_PER_SUB = 640
NC, NS = 2, 16

D_IN = 128
D_HID = 64
D_L2 = 48
D_OUT = 40

_sc_mesh = plsc.VectorSubcoreMesh(
    core_axis_name="c", subcore_axis_name="s", num_cores=NC, num_subcores=NS
)


def _make_sc_segsum(D, with_deg, _RING, hbm_frac=0):
    out_type = [jax.ShapeDtypeStruct((NC, N_PAD, D), jnp.float32)]
    scratch = [
        pltpu.VMEM((W_PER_SUB, 128), jnp.int32),
        pltpu.VMEM((W_PER_SUB, 128), jnp.int32),
        pltpu.VMEM((_RING, 128, D), jnp.float32),
        pltpu.VMEM_SHARED((N_PAD, D), jnp.float32),
        pltpu.VMEM_SHARED((N_PAD, D), jnp.float32),
    ]
    scratch += [pltpu.SemaphoreType.DMA] * (_RING + 6)
    if with_deg:
        out_type.append(jax.ShapeDtypeStruct((NC, N_PAD, 16), jnp.float32))
        scratch += [
            pltpu.VMEM((128, 16), jnp.float32),
            pltpu.VMEM_SHARED((N_PAD, 16), jnp.float32),
        ]

    def body(*refs):
        if with_deg:
            (table, srcm, dstm, zD, z16, ones_h,
             acc_out, deg_out,
             sidx, didx, rows, acc_sh, table_sh, *rest) = refs
            gsems = rest[:_RING]
            isems = rest[_RING:_RING + 6]
            ones_v, deg_sh = rest[_RING + 6:]
        else:
            (table, srcm, dstm, zD,
             acc_out,
             sidx, didx, rows, acc_sh, table_sh, *rest) = refs
            gsems = rest[:_RING]
            isems = rest[_RING:_RING + 6]

        cid = lax.axis_index("c")
        sid = lax.axis_index("s")
        wid = cid * NS + sid

        r0 = sid * ROWS_PER_SUB
        base = wid * W_PER_SUB
        ih = [
            pltpu.async_copy(zD.at[pl.ds(r0, ROWS_PER_SUB)],
                             acc_sh.at[pl.ds(r0, ROWS_PER_SUB)], isems[0]),
            pltpu.async_copy(table.at[pl.ds(r0, ROWS_PER_SUB)],
                             table_sh.at[pl.ds(r0, ROWS_PER_SUB)], isems[1]),
            pltpu.async_copy(srcm.at[pl.ds(base, W_PER_SUB)], sidx, isems[2]),
            pltpu.async_copy(dstm.at[pl.ds(base, W_PER_SUB)], didx, isems[3]),
        ]
        if with_deg:
            ih.append(pltpu.async_copy(z16.at[pl.ds(r0, ROWS_PER_SUB)],
                                       deg_sh.at[pl.ds(r0, ROWS_PER_SUB)],
                                       isems[4]))
            ih.append(pltpu.async_copy(ones_h, ones_v, isems[5]))
        for h in ih:
            h.wait()
        plsc.subcore_barrier()

        @pl.loop(0, W_PER_SUB, step=_RING)
        def _(j):
            gh = [
                pltpu.async_copy(
                    (table if (hbm_frac and k % hbm_frac == 0)
                     else table_sh).at[sidx.at[j + k]],
                    rows.at[k], gsems[k])
                for k in range(_RING)
            ]
            for k in range(_RING):
                gh[k].wait()
                pltpu.sync_copy(rows.at[k], acc_sh.at[didx.at[j + k]],
                                add=True)
                if with_deg:
                    pltpu.sync_copy(ones_v, deg_sh.at[didx.at[j + k]],
                                    add=True)

        plsc.subcore_barrier()

        oh = [pltpu.async_copy(acc_sh.at[pl.ds(r0, ROWS_PER_SUB)],
                               acc_out.at[cid, pl.ds(r0, ROWS_PER_SUB)],
                               isems[0])]
        if with_deg:
            oh.append(pltpu.async_copy(deg_sh.at[pl.ds(r0, ROWS_PER_SUB)],
                                       deg_out.at[cid, pl.ds(r0, ROWS_PER_SUB)],
                                       isems[1]))
        for h in oh:
            h.wait()

    return pl.kernel(
        body,
        out_type=tuple(out_type) if with_deg else out_type[0],
        mesh=_sc_mesh,
        scratch_types=scratch,
        compiler_params=pltpu.CompilerParams(use_tc_tiling_on_sc=False),
    )


_BLK = 1280


def _mm_body(x_ref, w_ref, b_ref, o1_ref, o2_ref):
    t = (jnp.dot(x_ref[...], w_ref[...], preferred_element_type=jnp.float32)
         + b_ref[...])
    o1_ref[...] = t[:, :D_HID]
    o2_ref[...] = t[:, D_HID:]


def _tc_matmul(xp, W, b):
    K = xp.shape[1]
    return pl.pallas_call(
        _mm_body,
        grid=(N_PAD // _BLK,),
        in_specs=[
            pl.BlockSpec((_BLK, K), lambda i: (i, 0)),
            pl.BlockSpec((K, 128), lambda i: (0, 0)),
            pl.BlockSpec((1, 128), lambda i: (0, 0)),
        ],
        out_specs=[pl.BlockSpec((_BLK, D_HID), lambda i: (i, 0)),
                   pl.BlockSpec((_BLK, D_HID), lambda i: (i, 0))],
        out_shape=[jax.ShapeDtypeStruct((N_PAD, D_HID), jnp.float32),
                   jax.ShapeDtypeStruct((N_PAD, D_HID), jnp.float32)],
    )(xp, W, b)


def _mid_body(a0, a1, g0, g1, xr, w_ref, b_ref, o_ref, o2_ref):
    deg = g0[...][:, 0:1] + g1[...][:, 0:1]
    dinv = 1.0 / jnp.maximum(deg, 1.0)
    h = jnp.maximum((a0[...] + a1[...]) * dinv + xr[...], 0.0)
    t = (jnp.dot(h, w_ref[...], preferred_element_type=jnp.float32)
         + b_ref[...])
    o_ref[...] = t[:, :D_L2]
    o2_ref[...] = t[:, D_L2:D_L2 + D_OUT]


def _tc_mid(a0, a1, g0, g1, xr, W, b):
    return pl.pallas_call(
        _mid_body,
        grid=(N_PAD // _BLK,),
        in_specs=[
            pl.BlockSpec((_BLK, D_HID), lambda i: (i, 0)),
            pl.BlockSpec((_BLK, D_HID), lambda i: (i, 0)),
            pl.BlockSpec((_BLK, 16), lambda i: (i, 0)),
            pl.BlockSpec((_BLK, 16), lambda i: (i, 0)),
            pl.BlockSpec((_BLK, D_HID), lambda i: (i, 0)),
            pl.BlockSpec((D_HID, 128), lambda i: (0, 0)),
            pl.BlockSpec((1, 128), lambda i: (0, 0)),
        ],
        out_specs=[pl.BlockSpec((_BLK, D_L2), lambda i: (i, 0)),
                   pl.BlockSpec((_BLK, D_OUT), lambda i: (i, 0))],
        out_shape=[jax.ShapeDtypeStruct((N_PAD, D_L2), jnp.float32),
                   jax.ShapeDtypeStruct((N_PAD, D_OUT), jnp.float32)],
    )(a0, a1, g0, g1, xr, W, b)


def _final_body(a0, a1, g0, g1, hr, o_ref):
    deg = g0[...][:, 0:1] + g1[...][:, 0:1]
    dinv = 1.0 / jnp.maximum(deg, 1.0)
    m = (a0[...] + a1[...])[:, :D_OUT] * dinv + hr[...]
    z = m - jnp.max(m, axis=1, keepdims=True)
    o_ref[...] = z - jnp.log(jnp.sum(jnp.exp(z), axis=1, keepdims=True))


def _tc_final(a0, a1, g0, g1, hr):
    return pl.pallas_call(
        _final_body,
        grid=(N_PAD // _BLK,),
        in_specs=[
            pl.BlockSpec((_BLK, D_L2), lambda i: (i, 0)),
            pl.BlockSpec((_BLK, D_L2), lambda i: (i, 0)),
            pl.BlockSpec((_BLK, 16), lambda i: (i, 0)),
            pl.BlockSpec((_BLK, 16), lambda i: (i, 0)),
            pl.BlockSpec((_BLK, D_OUT), lambda i: (i, 0)),
        ],
        out_specs=pl.BlockSpec((_BLK, D_OUT), lambda i: (i, 0)),
        out_shape=jax.ShapeDtypeStruct((N_PAD, D_OUT), jnp.float32),
    )(a0, a1, g0, g1, hr)


_sc_seg1 = _make_sc_segsum(D_HID, with_deg=True, _RING=2)
_sc_seg2 = _make_sc_segsum(D_L2, with_deg=False, _RING=4, hbm_frac=4)


def kernel(x, edge_index, W1_l, b1, W1_r, W2_l, b2, W2_r):
    f32 = jnp.float32
    xp = jnp.pad(x, ((0, N_PAD - N_NODES), (0, 0)))
    src = edge_index[0]
    dst = edge_index[1]
    n_extra = E_PAD - N_EDGES
    pad_src = jnp.zeros((n_extra,), jnp.int32)
    pad_dst = N_NODES + (jnp.arange(n_extra, dtype=jnp.int32) % (N_PAD - N_NODES))
    srcm = jnp.concatenate([src, pad_src]).reshape(E_ROWS, 128)
    dstm = jnp.concatenate([dst, pad_dst]).reshape(E_ROWS, 128)

    W1cat = jnp.concatenate([W1_l.T, W1_r.T], axis=1)
    b1cat = jnp.concatenate([jnp.zeros((64,), f32), b1]).reshape(1, 128)
    W2l_pad = jnp.pad(W2_l, ((0, D_L2 - D_OUT), (0, 0)))
    W2cat = jnp.concatenate(
        [W2l_pad.T, W2_r.T, jnp.zeros((D_HID, 128 - D_L2 - D_OUT), f32)], axis=1
    )
    b2cat = jnp.concatenate(
        [jnp.zeros((D_L2,), f32), b2, jnp.zeros((128 - D_L2 - D_OUT,), f32)]
    ).reshape(1, 128)

    z64 = jnp.zeros((N_PAD, D_HID), f32)
    z48 = jnp.zeros((N_PAD, D_L2), f32)
    z16 = jnp.zeros((N_PAD, 16), f32)
    ones16 = jnp.ones((128, 16), f32)

    y1, xr = _tc_matmul(xp, W1cat, b1cat)
    acc1, deg = _sc_seg1(y1, srcm, dstm, z64, z16, ones16)

    y2, hr = _tc_mid(acc1[0], acc1[1], deg[0], deg[1], xr, W2cat, b2cat)
    acc2 = _sc_seg2(y2, srcm, dstm, z48)

    out = _tc_final(acc2[0], acc2[1], deg[0], deg[1], hr)
    return out[:N_NODES]


# speedup vs baseline: 1.0752x; 1.0752x over previous
import functools

import jax
import jax.numpy as jnp
from jax import lax
from jax.experimental import pallas as pl
from jax.experimental.pallas import tpu as pltpu
from jax.experimental.pallas import tpu_sc as plsc

N_NODES = 10000
N_PAD = 10240
N_EDGES = 320000
E_PAD = 327680
E_ROWS = 2560
W_PER_SUB = 80
ROWS_PER_SUB = 640
NC, NS = 2, 16

D_IN = 128
D_HID = 64
D_L2 = 48
D_OUT = 40

_sc_mesh = plsc.VectorSubcoreMesh(
    core_axis_name="c", subcore_axis_name="s", num_cores=NC, num_subcores=NS
)


def _make_sc_segsum(D, with_deg, _RING):
    out_type = [jax.ShapeDtypeStruct((NC, N_PAD, D), jnp.float32)]
    scratch = [
        pltpu.VMEM((W_PER_SUB, 128), jnp.int32),
        pltpu.VMEM((W_PER_SUB, 128), jnp.int32),
        pltpu.VMEM((_RING, 128, D), jnp.float32),
        pltpu.VMEM_SHARED((N_PAD, D), jnp.float32),
        pltpu.VMEM_SHARED((N_PAD, D), jnp.float32),
    ]
    scratch += [pltpu.SemaphoreType.DMA] * (_RING + 6)
    if with_deg:
        out_type.append(jax.ShapeDtypeStruct((NC, N_PAD, 16), jnp.float32))
        scratch += [
            pltpu.VMEM((128, 16), jnp.float32),
            pltpu.VMEM_SHARED((N_PAD, 16), jnp.float32),
        ]

    def body(*refs):
        if with_deg:
            (table, srcm, dstm, zD, z16, ones_h,
             acc_out, deg_out,
             sidx, didx, rows, acc_sh, table_sh, *rest) = refs
            gsems = rest[:_RING]
            isems = rest[_RING:_RING + 6]
            ones_v, deg_sh = rest[_RING + 6:]
        else:
            (table, srcm, dstm, zD,
             acc_out,
             sidx, didx, rows, acc_sh, table_sh, *rest) = refs
            gsems = rest[:_RING]
            isems = rest[_RING:_RING + 6]

        cid = lax.axis_index("c")
        sid = lax.axis_index("s")
        wid = cid * NS + sid

        r0 = sid * ROWS_PER_SUB
        base = wid * W_PER_SUB
        ih = [
            pltpu.async_copy(zD.at[pl.ds(r0, ROWS_PER_SUB)],
                             acc_sh.at[pl.ds(r0, ROWS_PER_SUB)], isems[0]),
            pltpu.async_copy(table.at[pl.ds(r0, ROWS_PER_SUB)],
                             table_sh.at[pl.ds(r0, ROWS_PER_SUB)], isems[1]),
            pltpu.async_copy(srcm.at[pl.ds(base, W_PER_SUB)], sidx, isems[2]),
            pltpu.async_copy(dstm.at[pl.ds(base, W_PER_SUB)], didx, isems[3]),
        ]
        if with_deg:
            ih.append(pltpu.async_copy(z16.at[pl.ds(r0, ROWS_PER_SUB)],
                                       deg_sh.at[pl.ds(r0, ROWS_PER_SUB)],
                                       isems[4]))
            ih.append(pltpu.async_copy(ones_h, ones_v, isems[5]))
        for h in ih:
            h.wait()
        plsc.subcore_barrier()

        @pl.loop(0, W_PER_SUB, step=_RING)
        def _(j):
            gh = [
                pltpu.async_copy(table_sh.at[sidx.at[j + k]], rows.at[k],
                                 gsems[k])
                for k in range(_RING)
            ]
            for k in range(_RING):
                gh[k].wait()
                pltpu.sync_copy(rows.at[k], acc_sh.at[didx.at[j + k]],
                                add=True)
                if with_deg:
                    pltpu.sync_copy(ones_v, deg_sh.at[didx.at[j + k]],
                                    add=True)

        plsc.subcore_barrier()

        oh = [pltpu.async_copy(acc_sh.at[pl.ds(r0, ROWS_PER_SUB)],
                               acc_out.at[cid, pl.ds(r0, ROWS_PER_SUB)],
                               isems[0])]
        if with_deg:
            oh.append(pltpu.async_copy(deg_sh.at[pl.ds(r0, ROWS_PER_SUB)],
                                       deg_out.at[cid, pl.ds(r0, ROWS_PER_SUB)],
                                       isems[1]))
        for h in oh:
            h.wait()

    return pl.kernel(
        body,
        out_type=tuple(out_type) if with_deg else out_type[0],
        mesh=_sc_mesh,
        scratch_types=scratch,
        compiler_params=pltpu.CompilerParams(use_tc_tiling_on_sc=False),
    )


_BLK = 1280


def _mm_body(x_ref, w_ref, b_ref, o1_ref, o2_ref):
    t = (jnp.dot(x_ref[...], w_ref[...], preferred_element_type=jnp.float32)
         + b_ref[...])
    o1_ref[...] = t[:, :D_HID]
    o2_ref[...] = t[:, D_HID:]


def _tc_matmul(xp, W, b):
    K = xp.shape[1]
    return pl.pallas_call(
        _mm_body,
        grid=(N_PAD // _BLK,),
        in_specs=[
            pl.BlockSpec((_BLK, K), lambda i: (i, 0)),
            pl.BlockSpec((K, 128), lambda i: (0, 0)),
            pl.BlockSpec((1, 128), lambda i: (0, 0)),
        ],
        out_specs=[pl.BlockSpec((_BLK, D_HID), lambda i: (i, 0)),
                   pl.BlockSpec((_BLK, D_HID), lambda i: (i, 0))],
        out_shape=[jax.ShapeDtypeStruct((N_PAD, D_HID), jnp.float32),
                   jax.ShapeDtypeStruct((N_PAD, D_HID), jnp.float32)],
    )(xp, W, b)


def _mid_body(a0, a1, g0, g1, xr, w_ref, b_ref, o_ref, o2_ref):
    deg = g0[...][:, 0:1] + g1[...][:, 0:1]
    dinv = 1.0 / jnp.maximum(deg, 1.0)
    h = jnp.maximum((a0[...] + a1[...]) * dinv + xr[...], 0.0)
    t = (jnp.dot(h, w_ref[...], preferred_element_type=jnp.float32)
         + b_ref[...])
    o_ref[...] = t[:, :D_L2]
    o2_ref[...] = t[:, D_L2:D_L2 + D_OUT]


def _tc_mid(a0, a1, g0, g1, xr, W, b):
    return pl.pallas_call(
        _mid_body,
        grid=(N_PAD // _BLK,),
        in_specs=[
            pl.BlockSpec((_BLK, D_HID), lambda i: (i, 0)),
            pl.BlockSpec((_BLK, D_HID), lambda i: (i, 0)),
            pl.BlockSpec((_BLK, 16), lambda i: (i, 0)),
            pl.BlockSpec((_BLK, 16), lambda i: (i, 0)),
            pl.BlockSpec((_BLK, D_HID), lambda i: (i, 0)),
            pl.BlockSpec((D_HID, 128), lambda i: (0, 0)),
            pl.BlockSpec((1, 128), lambda i: (0, 0)),
        ],
        out_specs=[pl.BlockSpec((_BLK, D_L2), lambda i: (i, 0)),
                   pl.BlockSpec((_BLK, D_OUT), lambda i: (i, 0))],
        out_shape=[jax.ShapeDtypeStruct((N_PAD, D_L2), jnp.float32),
                   jax.ShapeDtypeStruct((N_PAD, D_OUT), jnp.float32)],
    )(a0, a1, g0, g1, xr, W, b)


def _final_body(a0, a1, g0, g1, hr, o_ref):
    deg = g0[...][:, 0:1] + g1[...][:, 0:1]
    dinv = 1.0 / jnp.maximum(deg, 1.0)
    m = (a0[...] + a1[...])[:, :D_OUT] * dinv + hr[...]
    z = m - jnp.max(m, axis=1, keepdims=True)
    o_ref[...] = z - jnp.log(jnp.sum(jnp.exp(z), axis=1, keepdims=True))


def _tc_final(a0, a1, g0, g1, hr):
    return pl.pallas_call(
        _final_body,
        grid=(N_PAD // _BLK,),
        in_specs=[
            pl.BlockSpec((_BLK, D_L2), lambda i: (i, 0)),
            pl.BlockSpec((_BLK, D_L2), lambda i: (i, 0)),
            pl.BlockSpec((_BLK, 16), lambda i: (i, 0)),
            pl.BlockSpec((_BLK, 16), lambda i: (i, 0)),
            pl.BlockSpec((_BLK, D_OUT), lambda i: (i, 0)),
        ],
        out_specs=pl.BlockSpec((_BLK, D_OUT), lambda i: (i, 0)),
        out_shape=jax.ShapeDtypeStruct((N_PAD, D_OUT), jnp.float32),
    )(a0, a1, g0, g1, hr)


_sc_seg1 = _make_sc_segsum(D_HID, with_deg=True, _RING=2)
_sc_seg2 = _make_sc_segsum(D_L2, with_deg=False, _RING=4)


def kernel(x, edge_index, W1_l, b1, W1_r, W2_l, b2, W2_r):
    f32 = jnp.float32
    xp = jnp.pad(x, ((0, N_PAD - N_NODES), (0, 0)))
    src = edge_index[0]
    dst = edge_index[1]
    n_extra = E_PAD - N_EDGES
    pad_src = jnp.zeros((n_extra,), jnp.int32)
    pad_dst = N_NODES + (jnp.arange(n_extra, dtype=jnp.int32) % (N_PAD - N_NODES))
    srcm = jnp.concatenate([src, pad_src]).reshape(E_ROWS, 128)
    dstm = jnp.concatenate([dst, pad_dst]).reshape(E_ROWS, 128)

    W1cat = jnp.concatenate([W1_l.T, W1_r.T], axis=1)
    b1cat = jnp.concatenate([jnp.zeros((64,), f32), b1]).reshape(1, 128)
    W2l_pad = jnp.pad(W2_l, ((0, D_L2 - D_OUT), (0, 0)))
    W2cat = jnp.concatenate(
        [W2l_pad.T, W2_r.T, jnp.zeros((D_HID, 128 - D_L2 - D_OUT), f32)], axis=1
    )
    b2cat = jnp.concatenate(
        [jnp.zeros((D_L2,), f32), b2, jnp.zeros((128 - D_L2 - D_OUT,), f32)]
    ).reshape(1, 128)

    z64 = jnp.zeros((N_PAD, D_HID), f32)
    z48 = jnp.zeros((N_PAD, D_L2), f32)
    z16 = jnp.zeros((N_PAD, 16), f32)
    ones16 = jnp.ones((128, 16), f32)

    y1, xr = _tc_matmul(xp, W1cat, b1cat)
    acc1, deg = _sc_seg1(y1, srcm, dstm, z64, z16, ones16)

    y2, hr = _tc_mid(acc1[0], acc1[1], deg[0], deg[1], xr, W2cat, b2cat)
    acc2 = _sc_seg2(y2, srcm, dstm, z48)

    out = _tc_final(acc2[0], acc2[1], deg[0], deg[1], hr)
    return out[:N_NODES]


# speedup vs baseline: 1.1020x; 1.0249x over previous
import functools

import jax
import jax.numpy as jnp
from jax import lax
from jax.experimental import pallas as pl
from jax.experimental.pallas import tpu as pltpu
from jax.experimental.pallas import tpu_sc as plsc

N_NODES = 10000
N_PAD = 10240
N_EDGES = 320000
E_PAD = 327680
E_ROWS = 2560
W_PER_SUB = 80
ROWS_PER_SUB = 640
NC, NS = 2, 16

D_IN = 128
D_HID = 64
D_L2 = 48
D_OUT = 40

_sc_mesh = plsc.VectorSubcoreMesh(
    core_axis_name="c", subcore_axis_name="s", num_cores=NC, num_subcores=NS
)


def _make_sc_segsum(D, with_deg, _RING):
    out_type = [jax.ShapeDtypeStruct((NC, N_PAD, D), jnp.float32)]
    scratch = [
        pltpu.VMEM((W_PER_SUB, 128), jnp.int32),
        pltpu.VMEM((W_PER_SUB, 128), jnp.int32),
        pltpu.VMEM((_RING, 128, D), jnp.float32),
        pltpu.VMEM_SHARED((N_PAD, D), jnp.float32),
        pltpu.VMEM_SHARED((N_PAD, D), jnp.float32),
    ]
    scratch += [pltpu.SemaphoreType.DMA] * (_RING + 6)
    if with_deg:
        out_type.append(jax.ShapeDtypeStruct((NC, N_PAD, 16), jnp.float32))
        scratch += [
            pltpu.VMEM((128, 16), jnp.float32),
            pltpu.VMEM_SHARED((N_PAD, 16), jnp.float32),
        ]

    def body(*refs):
        if with_deg:
            (table, srcm, dstm, zD, z16, ones_h,
             acc_out, deg_out,
             sidx, didx, rows, acc_sh, table_sh, *rest) = refs
            gsems = rest[:_RING]
            isems = rest[_RING:_RING + 6]
            ones_v, deg_sh = rest[_RING + 6:]
        else:
            (table, srcm, dstm, zD,
             acc_out,
             sidx, didx, rows, acc_sh, table_sh, *rest) = refs
            gsems = rest[:_RING]
            isems = rest[_RING:_RING + 6]

        cid = lax.axis_index("c")
        sid = lax.axis_index("s")
        wid = cid * NS + sid

        r0 = sid * ROWS_PER_SUB
        base = wid * W_PER_SUB
        ih = [
            pltpu.async_copy(zD.at[pl.ds(r0, ROWS_PER_SUB)],
                             acc_sh.at[pl.ds(r0, ROWS_PER_SUB)], isems[0]),
            pltpu.async_copy(table.at[pl.ds(r0, ROWS_PER_SUB)],
                             table_sh.at[pl.ds(r0, ROWS_PER_SUB)], isems[1]),
            pltpu.async_copy(srcm.at[pl.ds(base, W_PER_SUB)], sidx, isems[2]),
            pltpu.async_copy(dstm.at[pl.ds(base, W_PER_SUB)], didx, isems[3]),
        ]
        if with_deg:
            ih.append(pltpu.async_copy(z16.at[pl.ds(r0, ROWS_PER_SUB)],
                                       deg_sh.at[pl.ds(r0, ROWS_PER_SUB)],
                                       isems[4]))
            ih.append(pltpu.async_copy(ones_h, ones_v, isems[5]))
        for h in ih:
            h.wait()
        plsc.subcore_barrier()

        @pl.loop(0, W_PER_SUB, step=_RING)
        def _(j):
            gh = [
                pltpu.async_copy(table_sh.at[sidx.at[j + k]], rows.at[k],
                                 gsems[k])
                for k in range(_RING)
            ]
            for k in range(_RING):
                gh[k].wait()
                pltpu.sync_copy(rows.at[k], acc_sh.at[didx.at[j + k]],
                                add=True)
                if with_deg:
                    pltpu.sync_copy(ones_v, deg_sh.at[didx.at[j + k]],
                                    add=True)

        plsc.subcore_barrier()

        oh = [pltpu.async_copy(acc_sh.at[pl.ds(r0, ROWS_PER_SUB)],
                               acc_out.at[cid, pl.ds(r0, ROWS_PER_SUB)],
                               isems[0])]
        if with_deg:
            oh.append(pltpu.async_copy(deg_sh.at[pl.ds(r0, ROWS_PER_SUB)],
                                       deg_out.at[cid, pl.ds(r0, ROWS_PER_SUB)],
                                       isems[1]))
        for h in oh:
            h.wait()

    return pl.kernel(
        body,
        out_type=tuple(out_type) if with_deg else out_type[0],
        mesh=_sc_mesh,
        scratch_types=scratch,
        compiler_params=pltpu.CompilerParams(use_tc_tiling_on_sc=False),
    )


_BLK = 2560


def _mm_body(x_ref, w_ref, b_ref, o1_ref, o2_ref):
    t = (jnp.dot(x_ref[...], w_ref[...], preferred_element_type=jnp.float32)
         + b_ref[...])
    o1_ref[...] = t[:, :D_HID]
    o2_ref[...] = t[:, D_HID:]


def _tc_matmul(xp, W, b):
    K = xp.shape[1]
    return pl.pallas_call(
        _mm_body,
        grid=(N_PAD // _BLK,),
        in_specs=[
            pl.BlockSpec((_BLK, K), lambda i: (i, 0)),
            pl.BlockSpec((K, 128), lambda i: (0, 0)),
            pl.BlockSpec((1, 128), lambda i: (0, 0)),
        ],
        out_specs=[pl.BlockSpec((_BLK, D_HID), lambda i: (i, 0)),
                   pl.BlockSpec((_BLK, D_HID), lambda i: (i, 0))],
        out_shape=[jax.ShapeDtypeStruct((N_PAD, D_HID), jnp.float32),
                   jax.ShapeDtypeStruct((N_PAD, D_HID), jnp.float32)],
    )(xp, W, b)


def _mid_body(a0, a1, g0, g1, xr, w_ref, b_ref, o_ref, o2_ref):
    deg = g0[...][:, 0:1] + g1[...][:, 0:1]
    dinv = 1.0 / jnp.maximum(deg, 1.0)
    h = jnp.maximum((a0[...] + a1[...]) * dinv + xr[...], 0.0)
    t = (jnp.dot(h, w_ref[...], preferred_element_type=jnp.float32)
         + b_ref[...])
    o_ref[...] = t[:, :D_L2]
    o2_ref[...] = t[:, D_L2:D_L2 + D_OUT]


def _tc_mid(a0, a1, g0, g1, xr, W, b):
    return pl.pallas_call(
        _mid_body,
        grid=(N_PAD // _BLK,),
        in_specs=[
            pl.BlockSpec((_BLK, D_HID), lambda i: (i, 0)),
            pl.BlockSpec((_BLK, D_HID), lambda i: (i, 0)),
            pl.BlockSpec((_BLK, 16), lambda i: (i, 0)),
            pl.BlockSpec((_BLK, 16), lambda i: (i, 0)),
            pl.BlockSpec((_BLK, D_HID), lambda i: (i, 0)),
            pl.BlockSpec((D_HID, 128), lambda i: (0, 0)),
            pl.BlockSpec((1, 128), lambda i: (0, 0)),
        ],
        out_specs=[pl.BlockSpec((_BLK, D_L2), lambda i: (i, 0)),
                   pl.BlockSpec((_BLK, D_OUT), lambda i: (i, 0))],
        out_shape=[jax.ShapeDtypeStruct((N_PAD, D_L2), jnp.float32),
                   jax.ShapeDtypeStruct((N_PAD, D_OUT), jnp.float32)],
    )(a0, a1, g0, g1, xr, W, b)


def _final_body(a0, a1, g0, g1, hr, o_ref):
    deg = g0[...][:, 0:1] + g1[...][:, 0:1]
    dinv = 1.0 / jnp.maximum(deg, 1.0)
    m = (a0[...] + a1[...])[:, :D_OUT] * dinv + hr[...]
    z = m - jnp.max(m, axis=1, keepdims=True)
    o_ref[...] = z - jnp.log(jnp.sum(jnp.exp(z), axis=1, keepdims=True))


def _tc_final(a0, a1, g0, g1, hr):
    return pl.pallas_call(
        _final_body,
        grid=(N_PAD // _BLK,),
        in_specs=[
            pl.BlockSpec((_BLK, D_L2), lambda i: (i, 0)),
            pl.BlockSpec((_BLK, D_L2), lambda i: (i, 0)),
            pl.BlockSpec((_BLK, 16), lambda i: (i, 0)),
            pl.BlockSpec((_BLK, 16), lambda i: (i, 0)),
            pl.BlockSpec((_BLK, D_OUT), lambda i: (i, 0)),
        ],
        out_specs=pl.BlockSpec((_BLK, D_OUT), lambda i: (i, 0)),
        out_shape=jax.ShapeDtypeStruct((N_PAD, D_OUT), jnp.float32),
    )(a0, a1, g0, g1, hr)


_sc_seg1 = _make_sc_segsum(D_HID, with_deg=True, _RING=2)
_sc_seg2 = _make_sc_segsum(D_L2, with_deg=False, _RING=4)


def kernel(x, edge_index, W1_l, b1, W1_r, W2_l, b2, W2_r):
    f32 = jnp.float32
    xp = jnp.pad(x, ((0, N_PAD - N_NODES), (0, 0)))
    src = edge_index[0]
    dst = edge_index[1]
    n_extra = E_PAD - N_EDGES
    pad_src = jnp.zeros((n_extra,), jnp.int32)
    pad_dst = N_NODES + (jnp.arange(n_extra, dtype=jnp.int32) % (N_PAD - N_NODES))
    srcm = jnp.concatenate([src, pad_src]).reshape(E_ROWS, 128)
    dstm = jnp.concatenate([dst, pad_dst]).reshape(E_ROWS, 128)

    W1cat = jnp.concatenate([W1_l.T, W1_r.T], axis=1)
    b1cat = jnp.concatenate([jnp.zeros((64,), f32), b1]).reshape(1, 128)
    W2l_pad = jnp.pad(W2_l, ((0, D_L2 - D_OUT), (0, 0)))
    W2cat = jnp.concatenate(
        [W2l_pad.T, W2_r.T, jnp.zeros((D_HID, 128 - D_L2 - D_OUT), f32)], axis=1
    )
    b2cat = jnp.concatenate(
        [jnp.zeros((D_L2,), f32), b2, jnp.zeros((128 - D_L2 - D_OUT,), f32)]
    ).reshape(1, 128)

    z64 = jnp.zeros((N_PAD, D_HID), f32)
    z48 = jnp.zeros((N_PAD, D_L2), f32)
    z16 = jnp.zeros((N_PAD, 16), f32)
    ones16 = jnp.ones((128, 16), f32)

    y1, xr = _tc_matmul(xp, W1cat, b1cat)
    acc1, deg = _sc_seg1(y1, srcm, dstm, z64, z16, ones16)

    y2, hr = _tc_mid(acc1[0], acc1[1], deg[0], deg[1], xr, W2cat, b2cat)
    acc2 = _sc_seg2(y2, srcm, dstm, z48)

    out = _tc_final(acc2[0], acc2[1], deg[0], deg[1], hr)
    return out[:N_NODES]
